# Initial kernel scaffold; baseline (speedup 1.0000x reference)
import jax
import jax.numpy as jnp
from jax.experimental import pallas as pl


def kernel(x, edge_index, edge_weight, W, b):
    raise NotImplementedError("write your pallas kernel here")


# speedup vs baseline: 2.7395x; 2.7395x over previous
import functools

import jax
import jax.numpy as jnp
from jax import lax
from jax.experimental import pallas as pl
from jax.experimental.pallas import tpu as pltpu
from jax.experimental.pallas import tpu_sc as plsc

N_NODES = 10000
D = 128
N_EDGES = 320000

NW = 32
EDGES_PER_TILE = 10240
E_PAD = NW * EDGES_PER_TILE
CHUNK = 64
N_CHUNKS = EDGES_PER_TILE // CHUNK
ROWS_PER_TILE = 624
TAIL_ROWS = N_NODES - 16 * ROWS_PER_TILE
ZROWS = 104

_mesh = plsc.VectorSubcoreMesh(core_axis_name="c", subcore_axis_name="s")


@functools.partial(
    pl.kernel,
    mesh=_mesh,
    out_type=jax.ShapeDtypeStruct((2, N_NODES, D), jnp.float32),
    scratch_types=[
        pltpu.VMEM_SHARED((N_NODES, D), jnp.float32),
        pltpu.VMEM((ZROWS, D), jnp.float32),
        pltpu.VMEM((CHUNK,), jnp.int32),
        pltpu.VMEM((CHUNK,), jnp.int32),
        pltpu.VMEM((CHUNK,), jnp.float32),
        pltpu.VMEM((CHUNK, D), jnp.float32),
        pltpu.SemaphoreType.DMA,
    ],
)
def _sc_aggregate(x_hbm, src_hbm, dst_hbm, w_hbm, out_hbm,
                  acc, zbuf, src_b, dst_b, w_b, rows, sem):
    c = lax.axis_index("c")
    s = lax.axis_index("s")
    wid = c * 16 + s

    zero16 = jnp.zeros((16,), jnp.float32)
    for i in range(ZROWS):
        for j in range(D // 16):
            zbuf[i, pl.ds(j * 16, 16)] = zero16
    for r in range(ROWS_PER_TILE // ZROWS):
        pltpu.sync_copy(zbuf, acc.at[pl.ds(s * ROWS_PER_TILE + r * ZROWS, ZROWS)])

    @pl.when(s == 15)
    def _zero_tail():
        pltpu.sync_copy(zbuf.at[pl.ds(0, TAIL_ROWS)],
                        acc.at[pl.ds(16 * ROWS_PER_TILE, TAIL_ROWS)])

    plsc.subcore_barrier()

    def chunk_body(k, carry):
        base = wid * EDGES_PER_TILE + k * CHUNK
        pltpu.sync_copy(src_hbm.at[pl.ds(base, CHUNK)], src_b)
        pltpu.sync_copy(dst_hbm.at[pl.ds(base, CHUNK)], dst_b)
        pltpu.sync_copy(w_hbm.at[pl.ds(base, CHUNK)], w_b)
        pltpu.async_copy(x_hbm.at[src_b], rows, sem).wait()
        for g in range(CHUNK // 16):
            wv = w_b[pl.ds(g * 16, 16)]
            for t in range(16):
                i = g * 16 + t
                wb = jnp.broadcast_to(wv[t], (16,))
                for j in range(D // 16):
                    sl = pl.ds(j * 16, 16)
                    rows[i, sl] = rows[i, sl] * wb
        pltpu.sync_copy(rows, acc.at[dst_b], add=True)
        return carry

    lax.fori_loop(0, N_CHUNKS, chunk_body, 0)
    plsc.subcore_barrier()

    pltpu.sync_copy(acc.at[pl.ds(s * ROWS_PER_TILE, ROWS_PER_TILE)],
                    out_hbm.at[c, pl.ds(s * ROWS_PER_TILE, ROWS_PER_TILE)])

    @pl.when(s == 15)
    def _dump_tail():
        pltpu.sync_copy(acc.at[pl.ds(16 * ROWS_PER_TILE, TAIL_ROWS)],
                        out_hbm.at[c, pl.ds(16 * ROWS_PER_TILE, TAIL_ROWS)])


_BM = 2000


def _mm_body(p_ref, wt_ref, b_ref, o_ref):
    a = p_ref[0] + p_ref[1]
    o_ref[...] = (
        jnp.dot(a, wt_ref[...], preferred_element_type=jnp.float32) + b_ref[...]
    )


def _linear(partials, wt, b2):
    return pl.pallas_call(
        _mm_body,
        grid=(N_NODES // _BM,),
        in_specs=[
            pl.BlockSpec((2, _BM, D), lambda i: (0, i, 0)),
            pl.BlockSpec((D, D), lambda i: (0, 0)),
            pl.BlockSpec((1, D), lambda i: (0, 0)),
        ],
        out_specs=pl.BlockSpec((_BM, D), lambda i: (i, 0)),
        out_shape=jax.ShapeDtypeStruct((N_NODES, D), jnp.float32),
    )(partials, wt, b2)


def kernel(x, edge_index, edge_weight, W, b):
    dst = edge_index[0].astype(jnp.int32)
    src = edge_index[1].astype(jnp.int32)
    npad = E_PAD - N_EDGES
    src_p = jnp.concatenate([src, jnp.zeros((npad,), jnp.int32)])
    dst_p = jnp.concatenate([dst, jnp.zeros((npad,), jnp.int32)])
    w_p = jnp.concatenate([edge_weight, jnp.zeros((npad,), jnp.float32)])
    partials = _sc_aggregate(x, src_p, dst_p, w_p)
    return _linear(partials, W.T, b.reshape(1, D))
